# baseline (speedup 1.0000x reference)
import functools

import jax
import jax.numpy as jnp
from jax import lax
from jax.experimental import pallas as pl
from jax.experimental.pallas import tpu as pltpu
from jax.experimental.pallas import tpu_sc as plsc

EMBED_DIM = 64
EMB_SCALE = float(EMBED_DIM) ** 0.5
NUM_CORES = 2
NUM_SUBCORES = 16
NUM_WORKERS = NUM_CORES * NUM_SUBCORES
CHUNK = 128
LANES = 16


def _emb_body(idx_hbm, table_hbm, out_hbm, idx_v, rows_v, sem):
    n_chunks_w = idx_v.shape[0]
    wid = lax.axis_index("s") * NUM_CORES + lax.axis_index("c")
    crow = wid * n_chunks_w
    pltpu.sync_copy(idx_hbm.at[pl.ds(crow, n_chunks_w)], idx_v)

    def chunk_body(c, _):
        pltpu.async_copy(table_hbm.at[idx_v.at[c]], rows_v, sem).wait()

        def scale_body(r, _):
            for j in range(EMBED_DIM // LANES):
                sl = pl.ds(j * LANES, LANES)
                rows_v[r, sl] = rows_v[r, sl] * EMB_SCALE
            return ()

        lax.fori_loop(0, CHUNK, scale_body, (), unroll=2)
        pltpu.sync_copy(
            rows_v, out_hbm.at[pl.ds((crow + c) * CHUNK, CHUNK)])
        return ()

    lax.fori_loop(0, n_chunks_w, chunk_body, ())


def kernel(inputs, emb_weight):
    b, l = inputs.shape
    n = b * l
    assert n % (NUM_WORKERS * CHUNK) == 0
    n_chunks_w = n // (NUM_WORKERS * CHUNK)
    idx = inputs.reshape(n // CHUNK, CHUNK)

    mesh = plsc.VectorSubcoreMesh(core_axis_name="c", subcore_axis_name="s")
    emb = functools.partial(
        pl.kernel,
        mesh=mesh,
        out_type=jax.ShapeDtypeStruct((n, EMBED_DIM), jnp.float32),
        compiler_params=pltpu.CompilerParams(use_tc_tiling_on_sc=False),
        scratch_types=[
            pltpu.VMEM((n_chunks_w, CHUNK), jnp.int32),
            pltpu.VMEM((CHUNK, EMBED_DIM), jnp.float32),
            pltpu.SemaphoreType.DMA,
        ],
    )(_emb_body)
    out = emb(idx, emb_weight)
    return out.reshape(b, l, EMBED_DIM)


# speedup vs baseline: 1.1573x; 1.1573x over previous
import functools

import jax
import jax.numpy as jnp
from jax import lax
from jax.experimental import pallas as pl
from jax.experimental.pallas import tpu as pltpu
from jax.experimental.pallas import tpu_sc as plsc

EMBED_DIM = 64
EMB_SCALE = float(EMBED_DIM) ** 0.5
NUM_CORES = 2
NUM_SUBCORES = 16
NUM_WORKERS = NUM_CORES * NUM_SUBCORES
CHUNK = 128
NBUF = 8
LANES = 16


def _scale_rows(rows_v, b):
    def scale_body(r, _):
        for j in range(EMBED_DIM // LANES):
            sl = pl.ds(j * LANES, LANES)
            rows_v[b, r, sl] = rows_v[b, r, sl] * EMB_SCALE
        return ()

    lax.fori_loop(0, CHUNK, scale_body, (), unroll=4)


def _emb_body(idx_hbm, table_hbm, out_hbm, idx_v, rows_v, gsems, ssems):
    n_chunks_w = idx_v.shape[0]
    wid = lax.axis_index("s") * NUM_CORES + lax.axis_index("c")
    crow = wid * n_chunks_w
    pltpu.sync_copy(idx_hbm.at[pl.ds(crow, n_chunks_w)], idx_v)

    def start_gather(b, c):
        return pltpu.async_copy(
            table_hbm.at[idx_v.at[c]], rows_v.at[b], gsems.at[b])

    def start_store(b, c):
        return pltpu.async_copy(
            rows_v.at[b], out_hbm.at[pl.ds((crow + c) * CHUNK, CHUNK)],
            ssems.at[b])

    for b in range(NBUF):
        start_gather(b, b)

    n_groups = n_chunks_w // NBUF

    def full_group(g, _):
        c0 = g * NBUF
        for b in range(NBUF):
            c = c0 + b
            pltpu.make_async_copy(
                table_hbm.at[idx_v.at[c]], rows_v.at[b], gsems.at[b]).wait()
            _scale_rows(rows_v, b)
            start_store(b, c)
        for b in range(NBUF):
            c = c0 + b
            pltpu.make_async_copy(
                rows_v.at[b],
                out_hbm.at[pl.ds((crow + c) * CHUNK, CHUNK)],
                ssems.at[b]).wait()
            start_gather(b, c + NBUF)
        return ()

    lax.fori_loop(0, n_groups - 1, full_group, ())

    c0 = (n_groups - 1) * NBUF
    for b in range(NBUF):
        c = c0 + b
        pltpu.make_async_copy(
            table_hbm.at[idx_v.at[c]], rows_v.at[b], gsems.at[b]).wait()
        _scale_rows(rows_v, b)
        start_store(b, c)
    for b in range(NBUF):
        c = c0 + b
        pltpu.make_async_copy(
            rows_v.at[b],
            out_hbm.at[pl.ds((crow + c) * CHUNK, CHUNK)],
            ssems.at[b]).wait()


def kernel(inputs, emb_weight):
    b, l = inputs.shape
    n = b * l
    assert n % (NUM_WORKERS * CHUNK) == 0
    n_chunks_w = n // (NUM_WORKERS * CHUNK)
    assert n_chunks_w % NBUF == 0
    idx = inputs.reshape(n // CHUNK, CHUNK)

    mesh = plsc.VectorSubcoreMesh(core_axis_name="c", subcore_axis_name="s")
    emb = functools.partial(
        pl.kernel,
        mesh=mesh,
        out_type=jax.ShapeDtypeStruct((n, EMBED_DIM), jnp.float32),
        compiler_params=pltpu.CompilerParams(use_tc_tiling_on_sc=False),
        scratch_types=[
            pltpu.VMEM((n_chunks_w, CHUNK), jnp.int32),
            pltpu.VMEM((NBUF, CHUNK, EMBED_DIM), jnp.float32),
            pltpu.SemaphoreType.DMA((NBUF,)),
            pltpu.SemaphoreType.DMA((NBUF,)),
        ],
    )(_emb_body)
    out = emb(idx, emb_weight)
    return out.reshape(b, l, EMBED_DIM)
